# Initial kernel scaffold; baseline (speedup 1.0000x reference)
import jax
import jax.numpy as jnp
from jax.experimental import pallas as pl


def kernel(X, Wg, W1, b1, W2, b2):
    raise NotImplementedError("write your pallas kernel here")


# speedup vs baseline: 1.1698x; 1.1698x over previous
import functools

import jax
import jax.numpy as jnp
from jax import lax
from jax.experimental import pallas as pl
from jax.experimental.pallas import tpu as pltpu
from jax.experimental.pallas import tpu_sc as plsc

N_EMBD = 1024
EXPERTS = 8
EXPERT_DIM = 2048
N_TOK = 4096

ROW_TILE = 128
NP_ROWS = 2 * N_TOK + EXPERTS * ROW_TILE
NUM_TILES = NP_ROWS // ROW_TILE

GATE_CHUNK = 256
GATE_STEPS = N_TOK // GATE_CHUNK

NW = 32
TOK_PER_W = N_TOK // NW
DISP_CHUNK = 64
CMB_CHUNK = 32


def _gate_body(x_ref, wg_ref, e0_ref, e1_ref, w0_ref, w1_ref,
               r0_ref, r1_ref, cnt_ref, carry_ref):
    i = pl.program_id(0)

    @pl.when(i == 0)
    def _():
        carry_ref[...] = jnp.zeros_like(carry_ref)

    x = x_ref[...]
    logits = jnp.dot(x, wg_ref[...], preferred_element_type=jnp.float32)
    iota_e = lax.broadcasted_iota(jnp.int32, logits.shape, 1)

    v0 = jnp.max(logits, axis=1, keepdims=True)
    e0 = jnp.min(jnp.where(logits == v0, iota_e, EXPERTS), axis=1,
                 keepdims=True)
    masked = jnp.where(iota_e == e0, -jnp.inf, logits)
    v1 = jnp.max(masked, axis=1, keepdims=True)
    e1 = jnp.min(jnp.where(masked == v1, iota_e, EXPERTS), axis=1,
                 keepdims=True)
    w0 = 1.0 / (1.0 + jnp.exp(v1 - v0))
    w1 = 1.0 - w0

    epair = jnp.concatenate([e0, e1], axis=0)
    oh = (epair == lax.broadcasted_iota(jnp.int32, (2 * GATE_CHUNK, EXPERTS),
                                        1)).astype(jnp.float32)
    ri = lax.broadcasted_iota(jnp.int32, (2 * GATE_CHUNK, 2 * GATE_CHUNK), 0)
    ci = lax.broadcasted_iota(jnp.int32, (2 * GATE_CHUNK, 2 * GATE_CHUNK), 1)
    tril = (ri > ci).astype(jnp.float32)
    ranks = jnp.dot(tril, oh, preferred_element_type=jnp.float32)
    ranks = ranks + carry_ref[...]
    rank_pair = jnp.sum(ranks * oh, axis=1, keepdims=True)

    carry_ref[...] = carry_ref[...] + jnp.sum(oh, axis=0, keepdims=True)

    e0_ref[...] = e0
    e1_ref[...] = e1
    w0_ref[...] = w0
    w1_ref[...] = w1
    r0_ref[...] = rank_pair[:GATE_CHUNK].astype(jnp.int32)
    r1_ref[...] = rank_pair[GATE_CHUNK:].astype(jnp.int32)
    cnt_ref[...] = carry_ref[...]


def _gate(x_flat, Wg):
    f32 = jnp.float32
    i32 = jnp.int32
    outs = pl.pallas_call(
        _gate_body,
        grid=(GATE_STEPS,),
        in_specs=[
            pl.BlockSpec((GATE_CHUNK, N_EMBD), lambda i: (i, 0)),
            pl.BlockSpec((N_EMBD, EXPERTS), lambda i: (0, 0)),
        ],
        out_specs=[
            pl.BlockSpec((GATE_CHUNK, 1), lambda i: (i, 0)),
            pl.BlockSpec((GATE_CHUNK, 1), lambda i: (i, 0)),
            pl.BlockSpec((GATE_CHUNK, 1), lambda i: (i, 0)),
            pl.BlockSpec((GATE_CHUNK, 1), lambda i: (i, 0)),
            pl.BlockSpec((GATE_CHUNK, 1), lambda i: (i, 0)),
            pl.BlockSpec((GATE_CHUNK, 1), lambda i: (i, 0)),
            pl.BlockSpec((1, EXPERTS), lambda i: (0, 0)),
        ],
        out_shape=[
            jax.ShapeDtypeStruct((N_TOK, 1), i32),
            jax.ShapeDtypeStruct((N_TOK, 1), i32),
            jax.ShapeDtypeStruct((N_TOK, 1), f32),
            jax.ShapeDtypeStruct((N_TOK, 1), f32),
            jax.ShapeDtypeStruct((N_TOK, 1), i32),
            jax.ShapeDtypeStruct((N_TOK, 1), i32),
            jax.ShapeDtypeStruct((1, EXPERTS), f32),
        ],
        scratch_shapes=[pltpu.VMEM((1, EXPERTS), f32)],
    )(x_flat, Wg)
    return outs


_SC_MESH = plsc.VectorSubcoreMesh(core_axis_name="c", subcore_axis_name="s")


@functools.partial(
    pl.kernel,
    out_type=jax.ShapeDtypeStruct((NP_ROWS, N_EMBD), jnp.float32),
    mesh=_SC_MESH,
    scratch_types=[
        pltpu.VMEM((DISP_CHUNK, N_EMBD), jnp.float32),
        pltpu.VMEM((DISP_CHUNK,), jnp.int32),
        pltpu.VMEM((DISP_CHUNK,), jnp.int32),
        pltpu.SemaphoreType.DMA,
    ],
)
def _dispatch(x_hbm, pos0_hbm, pos1_hbm, xe_hbm, xbuf, i0, i1, sem):
    wid = lax.axis_index("s") * 2 + lax.axis_index("c")
    t0 = wid * TOK_PER_W

    def chunk(ci, carry):
        b = t0 + ci * DISP_CHUNK
        pltpu.sync_copy(x_hbm.at[pl.ds(b, DISP_CHUNK)], xbuf)
        pltpu.sync_copy(pos0_hbm.at[pl.ds(b, DISP_CHUNK)], i0)
        pltpu.sync_copy(pos1_hbm.at[pl.ds(b, DISP_CHUNK)], i1)
        cp0 = pltpu.async_copy(xbuf, xe_hbm.at[i0], sem)
        cp1 = pltpu.async_copy(xbuf, xe_hbm.at[i1], sem)
        cp0.wait()
        cp1.wait()
        return carry

    lax.fori_loop(0, TOK_PER_W // DISP_CHUNK, chunk, 0)


def _ffn_body(te_ref, xe_ref, w1_ref, b1_ref, w2_ref, b2_ref, y_ref):
    del te_ref
    x = xe_ref[...].astype(jnp.bfloat16)
    h = jnp.dot(x, w1_ref[0], preferred_element_type=jnp.float32)
    h = jax.nn.gelu(h + b1_ref[0])
    y = jnp.dot(h.astype(jnp.bfloat16), w2_ref[0],
                preferred_element_type=jnp.float32)
    y_ref[...] = y + b2_ref[0]


def _ffn(tile_e, xe, W1, b1, W2, b2):
    grid_spec = pltpu.PrefetchScalarGridSpec(
        num_scalar_prefetch=1,
        grid=(NUM_TILES,),
        in_specs=[
            pl.BlockSpec((ROW_TILE, N_EMBD), lambda i, te: (i, 0)),
            pl.BlockSpec((1, N_EMBD, EXPERT_DIM), lambda i, te: (te[i], 0, 0)),
            pl.BlockSpec((1, 1, EXPERT_DIM), lambda i, te: (te[i], 0, 0)),
            pl.BlockSpec((1, EXPERT_DIM, N_EMBD), lambda i, te: (te[i], 0, 0)),
            pl.BlockSpec((1, 1, N_EMBD), lambda i, te: (te[i], 0, 0)),
        ],
        out_specs=pl.BlockSpec((ROW_TILE, N_EMBD), lambda i, te: (i, 0)),
    )
    return pl.pallas_call(
        _ffn_body,
        grid_spec=grid_spec,
        out_shape=jax.ShapeDtypeStruct((NP_ROWS, N_EMBD), jnp.float32),
    )(tile_e, xe, W1, b1, W2, b2)


@functools.partial(
    pl.kernel,
    out_type=[
        jax.ShapeDtypeStruct((N_TOK, N_EMBD), jnp.float32),
        jax.ShapeDtypeStruct((N_TOK, N_EMBD), jnp.float32),
    ],
    mesh=_SC_MESH,
    scratch_types=[
        pltpu.VMEM((CMB_CHUNK, N_EMBD), jnp.float32),
        pltpu.VMEM((CMB_CHUNK, N_EMBD), jnp.float32),
        pltpu.VMEM((CMB_CHUNK,), jnp.int32),
        pltpu.VMEM((CMB_CHUNK,), jnp.int32),
        pltpu.SemaphoreType.DMA,
    ],
)
def _gather2(y_hbm, pos0_hbm, pos1_hbm, y0_hbm, y1_hbm,
             g0, g1, i0, i1, sem):
    wid = lax.axis_index("s") * 2 + lax.axis_index("c")
    t0 = wid * TOK_PER_W

    def chunk(ci, carry):
        b = t0 + ci * CMB_CHUNK
        pltpu.sync_copy(pos0_hbm.at[pl.ds(b, CMB_CHUNK)], i0)
        pltpu.sync_copy(pos1_hbm.at[pl.ds(b, CMB_CHUNK)], i1)
        cp0 = pltpu.async_copy(y_hbm.at[i0], g0, sem)
        cp1 = pltpu.async_copy(y_hbm.at[i1], g1, sem)
        cp0.wait()
        cp1.wait()
        pltpu.sync_copy(g0, y0_hbm.at[pl.ds(b, CMB_CHUNK)])
        pltpu.sync_copy(g1, y1_hbm.at[pl.ds(b, CMB_CHUNK)])
        return carry

    lax.fori_loop(0, TOK_PER_W // CMB_CHUNK, chunk, 0)


WSUM_CHUNK = 256


def _wsum_body(y0_ref, y1_ref, w0_ref, w1_ref, o_ref):
    o_ref[...] = (w0_ref[...] * y0_ref[...] + w1_ref[...] * y1_ref[...])


def _wsum(y0, y1, w0, w1):
    return pl.pallas_call(
        _wsum_body,
        grid=(N_TOK // WSUM_CHUNK,),
        in_specs=[
            pl.BlockSpec((WSUM_CHUNK, N_EMBD), lambda i: (i, 0)),
            pl.BlockSpec((WSUM_CHUNK, N_EMBD), lambda i: (i, 0)),
            pl.BlockSpec((WSUM_CHUNK, 1), lambda i: (i, 0)),
            pl.BlockSpec((WSUM_CHUNK, 1), lambda i: (i, 0)),
        ],
        out_specs=pl.BlockSpec((WSUM_CHUNK, N_EMBD), lambda i: (i, 0)),
        out_shape=jax.ShapeDtypeStruct((N_TOK, N_EMBD), jnp.float32),
    )(y0, y1, w0, w1)


def kernel(X, Wg, W1, b1, W2, b2):
    Bx, Tx, C = X.shape
    x_flat = X.reshape(-1, C)

    e0, e1, w0, w1, r0, r1, cnt = _gate(x_flat, Wg)
    e0 = e0.reshape(N_TOK)
    e1 = e1.reshape(N_TOK)
    counts = cnt.reshape(EXPERTS).astype(jnp.int32)

    padded = ((counts + ROW_TILE - 1) // ROW_TILE) * ROW_TILE
    base = jnp.concatenate(
        [jnp.zeros((1,), jnp.int32), jnp.cumsum(padded)[:-1]])
    pos0 = base[e0] + r0.reshape(N_TOK)
    pos1 = base[e1] + r1.reshape(N_TOK)
    tile_starts = jnp.arange(NUM_TILES, dtype=jnp.int32) * ROW_TILE
    tile_e = (jnp.searchsorted(base, tile_starts, side="right")
              .astype(jnp.int32) - 1)

    xe = _dispatch(x_flat, pos0, pos1)
    y = _ffn(tile_e, xe, W1.astype(jnp.bfloat16),
             b1.reshape(EXPERTS, 1, EXPERT_DIM),
             W2.astype(jnp.bfloat16),
             b2.reshape(EXPERTS, 1, N_EMBD))
    y0, y1 = _gather2(y, pos0, pos1)
    out = _wsum(y0, y1, w0, w1)
    return out.reshape(Bx, Tx, C)
